# baseline (speedup 1.0000x reference)
import jax
import jax.numpy as jnp
from jax import lax
from jax.experimental import pallas as pl
from jax.experimental.pallas import tpu as pltpu
from jax.experimental.pallas import tpu_sc as plsc

N_ROWS = 1_000_000
D = 32
B = 16384

NC, NS, L = 2, 16, 16
NW = NC * NS
FB = D // 8
NTC_FULL = N_ROWS // 128
TAIL0 = NTC_FULL * 128
NKMAX = NTC_FULL // NW + 1
TPC = 2
NCHUNK_MAX = (NKMAX + TPC - 1) // TPC
CAP = 96
TCAP = 64
TRASH = 2 * B
GROWS = 2 * B + 1
NSLOT = CAP // 16
ROWOFF = [(d // 8) * 8 + d % 8 for d in range(D)]

SCAN_IT = 2 * B // L // 2

CBLK = 16384
NBLK = -(-N_ROWS // CBLK)


def _p1_body(zt_hbm, yt_hbm, u_hbm, i_hbm, g_hbm,
             ubuf, ibuf, hl, tl, hcnt, kscr, vmscr,
             zbuf, ybuf, ztail, ytail, rows_out, sem_in, sem_out):
    wid = lax.axis_index("s") * NC + lax.axis_index("c")
    nk = jnp.where(wid < NTC_FULL - NW * (NKMAX - 1), NKMAX, NKMAX - 1)
    nchunks = lax.div(nk + (TPC - 1), TPC)

    cu = pltpu.async_copy(u_hbm, ubuf, sem_in)
    ci = pltpu.async_copy(i_hbm, ibuf, sem_in)
    cu.wait()
    ci.wait()

    lanes = lax.iota(jnp.int32, L)
    zeros = jnp.zeros((L,), jnp.int32)
    for t in range(8):
        hcnt[pl.ds(t * L, L)] = zeros

    def scan(t, tcnt):
        for side, buf in ((0, ubuf), (1, ibuf)):
            cv = buf[pl.ds(t * L, L)]
            pay = side * B + t * L + lanes
            j = lax.shift_right_logical(cv, 7)
            mine = (cv < TAIL0) & ((j & (NW - 1)) == wid)
            k = lax.shift_right_logical(jnp.where(mine, j - wid, 0), 5)
            m = lax.shift_right_logical(k, 1)
            q = k & (TPC - 1)
            word = lax.shift_left(q, 23) | lax.shift_left(cv & 127, 16) | pay
            ks, ws, vmask = plsc.sort_key_val(m, word, mask=mine)
            ksafe = jnp.where(vmask, ks, 0)
            vm = jnp.where(vmask, 1, 0)
            kscr[...] = ksafe
            vmscr[...] = vm
            prev = plsc.load_gather(kscr, [jnp.maximum(lanes - 1, 0)])
            nxt = plsc.load_gather(kscr, [jnp.minimum(lanes + 1, L - 1)])
            vmn = plsc.load_gather(vmscr, [jnp.minimum(lanes + 1, L - 1)])
            is_start = (lanes == 0) | (ksafe != prev)
            is_end = vmask & ((lanes == L - 1) | (ksafe != nxt) | (vmn == 0))
            startpos = plsc.cummax(jnp.where(is_start, lanes, 0))
            rank = lanes - startpos
            basec = plsc.load_gather(hcnt, [ksafe])
            dst = ksafe * CAP + basec + rank
            plsc.store_scatter(hl, [dst], ws, mask=vmask)
            plsc.addupdate_scatter(hcnt, [ksafe], rank + 1, mask=is_end)
            tmask = (cv >= TAIL0) & (wid == NW - 1)
            tword = lax.shift_left(cv - TAIL0, 16) | pay
            toffs = plsc.cumsum(jnp.where(tmask, 1, 0))
            plsc.store_scatter(tl, [tcnt + toffs - 1], tword, mask=tmask)
            tcnt = tcnt + plsc.all_reduce_population_count(tmask)
        return tcnt

    tcnt = lax.fori_loop(0, SCAN_IT, scan, zeros)

    def chunk(m, carry):
        copies = []
        for q in range(TPC):
            jq = wid + NW * jnp.minimum(TPC * m + q, nk - 1)
            col0 = pl.multiple_of(jq * 128, 128)
            for i in range(FB):
                src = zt_hbm.at[pl.ds(8 * i, 8), pl.ds(col0, 128)]
                copies.append(pltpu.async_copy(
                    src, zbuf.at[pl.ds(q * 32 + i * 8, 8), :], sem_in))
                src = yt_hbm.at[pl.ds(8 * i, 8), pl.ds(col0, 128)]
                copies.append(pltpu.async_copy(
                    src, ybuf.at[pl.ds(q * 32 + i * 8, 8), :], sem_in))
        for cp in copies:
            cp.wait()

        n = jnp.max(plsc.load_gather(hcnt, [jnp.full((L,), m, jnp.int32)]))
        rup = (n + 15) & ~15
        plsc.store_scatter(hl, [m * CAP + n + lanes],
                           jnp.full((L,), TRASH, jnp.int32),
                           mask=lanes < rup - n)
        nsub = lax.shift_right_logical(rup, 4)

        def sub(s, c2):
            w_v = hl[pl.ds(m * CAP + s * L, L)]
            qv = lax.shift_right_logical(w_v, 23)
            lanev = lax.shift_right_logical(w_v, 16) & 127
            payv = w_v & 0xFFFF
            qrow = lax.shift_left(qv, 5)
            ssp = jnp.full((L,), s, jnp.int32)
            for d in range(D):
                rowv = qrow + ROWOFF[d]
                zv = plsc.load_gather(zbuf, [rowv, lanev])
                yv = plsc.load_gather(ybuf, [rowv, lanev])
                dsp = jnp.full((L,), d, jnp.int32)
                plsc.store_scatter(rows_out, [ssp, lanes, dsp], zv)
                plsc.store_scatter(rows_out, [ssp, lanes, dsp + D], yv)
            pltpu.async_copy(rows_out.at[s], g_hbm.at[payv], sem_out)
            return c2

        lax.fori_loop(0, nsub, sub, 0)

        def drain(s, c2):
            pltpu.make_async_copy(g_hbm.at[pl.ds(0, L), :],
                                  rows_out.at[s], sem_out).wait()
            return c2

        lax.fori_loop(0, nsub, drain, 0)
        return carry

    lax.fori_loop(0, nchunks, chunk, 0)

    copies = []
    for i in range(FB):
        copies.append(pltpu.async_copy(
            zt_hbm.at[pl.ds(8 * i, 8), pl.ds(TAIL0, 64)],
            ztail.at[pl.ds(i * 8, 8), :], sem_in))
        copies.append(pltpu.async_copy(
            yt_hbm.at[pl.ds(8 * i, 8), pl.ds(TAIL0, 64)],
            ytail.at[pl.ds(i * 8, 8), :], sem_in))
    for cp in copies:
        cp.wait()

    tn = jnp.max(tcnt)
    trup = (tn + 15) & ~15
    plsc.store_scatter(tl, [tn + lanes], jnp.full((L,), TRASH, jnp.int32),
                       mask=lanes < trup - tn)

    def tsub(s, c2):
        w_v = tl[pl.ds(s * L, L)]
        lanev = lax.shift_right_logical(w_v, 16) & 127
        payv = w_v & 0xFFFF
        for d in range(D):
            rowv = jnp.full((L,), ROWOFF[d], jnp.int32)
            zv = plsc.load_gather(ztail, [rowv, lanev])
            yv = plsc.load_gather(ytail, [rowv, lanev])
            dsp = jnp.full((L,), d, jnp.int32)
            plsc.store_scatter(rows_out, [zeros, lanes, dsp], zv)
            plsc.store_scatter(rows_out, [zeros, lanes, dsp + D], yv)
        pltpu.async_copy(rows_out.at[0], g_hbm.at[payv], sem_out).wait()
        return c2

    lax.fori_loop(0, lax.shift_right_logical(trup, 4), tsub, 0)


def _p2_body(g_hbm, r_hbm, out_hbm, gu, gi, rbuf, outv, sem):
    wid = lax.axis_index("s") * NC + lax.axis_index("c")
    lanes = lax.iota(jnp.int32, L)
    acc = jnp.zeros((L,), jnp.float32)
    for c in range(4):
        b0 = pl.multiple_of((wid * 4 + c) * 128, 128)
        cu = pltpu.async_copy(g_hbm.at[pl.ds(b0, 128), :], gu, sem)
        ci = pltpu.async_copy(g_hbm.at[pl.ds(B + b0, 128), :], gi, sem)
        cr = pltpu.async_copy(r_hbm.at[pl.ds(b0, 128)], rbuf, sem)
        cu.wait()
        ci.wait()
        cr.wait()

        def group(g, a):
            rows = g * L + lanes
            rhat = jnp.zeros((L,), jnp.float32)
            for d in range(2 * D):
                dsp = jnp.full((L,), d, jnp.int32)
                rhat = rhat + (plsc.load_gather(gu, [rows, dsp])
                               * plsc.load_gather(gi, [rows, dsp]))
            err = rbuf[pl.ds(g * L, L)] - rhat
            return a + err * err

        acc = lax.fori_loop(0, 8, group, acc)
    for t in range(8):
        outv[pl.ds(t * L, L)] = jnp.zeros((L,), jnp.float32)
    outv[pl.ds(0, L)] = acc
    pltpu.sync_copy(outv, out_hbm.at[wid])


def _build_p1():
    return pl.kernel(
        _p1_body,
        out_type=jax.ShapeDtypeStruct((GROWS, 128), jnp.float32),
        mesh=plsc.VectorSubcoreMesh(core_axis_name="c", subcore_axis_name="s",
                                    num_cores=NC, num_subcores=NS),
        compiler_params=pltpu.CompilerParams(needs_layout_passes=False,
                                             use_tc_tiling_on_sc=True),
        scratch_types=[
            pltpu.VMEM((B,), jnp.int32),
            pltpu.VMEM((B,), jnp.int32),
            pltpu.VMEM((NCHUNK_MAX * CAP,), jnp.int32),
            pltpu.VMEM((TCAP,), jnp.int32),
            pltpu.VMEM((128,), jnp.int32),
            pltpu.VMEM((L,), jnp.int32),
            pltpu.VMEM((L,), jnp.int32),
            pltpu.VMEM((TPC * 32, 128), jnp.float32),
            pltpu.VMEM((TPC * 32, 128), jnp.float32),
            pltpu.VMEM((32, 64), jnp.float32),
            pltpu.VMEM((32, 64), jnp.float32),
            pltpu.VMEM((NSLOT, L, 128), jnp.float32),
            pltpu.SemaphoreType.DMA,
            pltpu.SemaphoreType.DMA,
        ],
    )


def _build_p2():
    return pl.kernel(
        _p2_body,
        out_type=jax.ShapeDtypeStruct((NW, 128), jnp.float32),
        mesh=plsc.VectorSubcoreMesh(core_axis_name="c", subcore_axis_name="s",
                                    num_cores=NC, num_subcores=NS),
        compiler_params=pltpu.CompilerParams(needs_layout_passes=False,
                                             use_tc_tiling_on_sc=True),
        scratch_types=[
            pltpu.VMEM((128, 128), jnp.float32),
            pltpu.VMEM((128, 128), jnp.float32),
            pltpu.VMEM((128,), jnp.float32),
            pltpu.VMEM((128,), jnp.float32),
            pltpu.SemaphoreType.DMA,
        ],
    )


def _tc_body(yt_ref, p_ref, o_ref):
    b = pl.program_id(0)

    @pl.when(b == 0)
    def _():
        pcols = lax.broadcasted_iota(jnp.int32, (NW, 128), 1)
        o_ref[0, 0] = jnp.sum(jnp.where(pcols < L, p_ref[...], 0.0))

    yv = yt_ref[...]
    cols = b * CBLK + lax.broadcasted_iota(jnp.int32, (D, CBLK), 1)
    yv = jnp.where(cols < N_ROWS, yv, 0.0)
    o_ref[0, 0] += jnp.sum(yv * yv)


_tc_loss = pl.pallas_call(
    _tc_body,
    grid=(NBLK,),
    in_specs=[
        pl.BlockSpec((D, CBLK), lambda b: (0, b)),
        pl.BlockSpec((NW, 128), lambda b: (0, 0)),
    ],
    out_specs=pl.BlockSpec(memory_space=pltpu.SMEM),
    out_shape=jax.ShapeDtypeStruct((1, 1), jnp.float32),
)


def kernel(Z, Y, interaction):
    u = interaction[:, 0]
    i = interaction[:, 1]
    r = interaction[:, 2].astype(jnp.float32)
    zt = Z.T
    yt = Y.T
    g = _build_p1()(zt, yt, u, i)
    partials = _build_p2()(g, r)
    loss = _tc_loss(yt, partials)
    return loss[0, 0]
